# initial kernel scaffold (unmeasured)
import jax
import jax.numpy as jnp
from jax import lax
from jax.experimental import pallas as pl
from jax.experimental.pallas import tpu as pltpu

N_DEV = 32


def kernel(x, w_mat):
    m_glob, k_per = x.shape
    k_glob, n = w_mat.shape
    m_per = m_glob // N_DEV

    def body(x_ref, w_ref, out_ref, send_buf, comm_ref, local_sem,
             send_sems, recv_sems):
        my = lax.axis_index("i")

        send_buf[...] = x_ref[...].astype(jnp.bfloat16)

        diag = pltpu.make_async_copy(
            send_buf.at[pl.ds(my * m_per, m_per), :],
            comm_ref.at[:, pl.ds(my * k_per, k_per)],
            local_sem,
        )
        diag.start()

        for j in range(N_DEV):
            @pl.when(j != my)
            def _(j=j):
                pltpu.make_async_remote_copy(
                    src_ref=send_buf.at[pl.ds(j * m_per, m_per), :],
                    dst_ref=comm_ref.at[:, pl.ds(my * k_per, k_per)],
                    send_sem=send_sems.at[j],
                    recv_sem=recv_sems.at[my],
                    device_id=(j,),
                    device_id_type=pl.DeviceIdType.MESH,
                ).start()

        diag.wait()
        for j in range(N_DEV):
            @pl.when(j != my)
            def _(j=j):
                pltpu.make_async_remote_copy(
                    src_ref=send_buf.at[pl.ds(j * m_per, m_per), :],
                    dst_ref=comm_ref.at[:, pl.ds(j * k_per, k_per)],
                    send_sem=send_sems.at[j],
                    recv_sem=recv_sems.at[j],
                    device_id=(j,),
                    device_id_type=pl.DeviceIdType.MESH,
                ).wait_recv()

        n_groups = 4
        kg = k_glob // n_groups
        acc = jnp.zeros((m_per, n), jnp.float32)
        for g in range(n_groups):
            wg = w_ref[g * kg:(g + 1) * kg, :].astype(jnp.bfloat16)
            acc = acc + jnp.dot(
                comm_ref[:, g * kg:(g + 1) * kg], wg,
                preferred_element_type=jnp.float32,
            )
        out_ref[...] = acc * (1.0 / (1.0 + jnp.exp(-acc)))

        for j in range(N_DEV):
            @pl.when(j != my)
            def _(j=j):
                pltpu.make_async_remote_copy(
                    src_ref=send_buf.at[pl.ds(j * m_per, m_per), :],
                    dst_ref=comm_ref.at[:, pl.ds(my * k_per, k_per)],
                    send_sem=send_sems.at[j],
                    recv_sem=recv_sems.at[my],
                    device_id=(j,),
                    device_id_type=pl.DeviceIdType.MESH,
                ).wait_send()

    return pl.pallas_call(
        body,
        out_shape=jax.ShapeDtypeStruct((m_per, n), jnp.float32),
        in_specs=[
            pl.BlockSpec(memory_space=pltpu.VMEM),
            pl.BlockSpec(memory_space=pltpu.VMEM),
        ],
        out_specs=pl.BlockSpec(memory_space=pltpu.VMEM),
        scratch_shapes=[
            pltpu.VMEM((m_glob, k_per), jnp.bfloat16),
            pltpu.VMEM((m_per, k_glob), jnp.bfloat16),
            pltpu.SemaphoreType.DMA,
            pltpu.SemaphoreType.DMA((N_DEV,)),
            pltpu.SemaphoreType.DMA((N_DEV,)),
        ],
        compiler_params=pltpu.CompilerParams(collective_id=0),
    )(x, w_mat)


# baseline (device time: 42572 ns/iter reference)
import jax
import jax.numpy as jnp
from jax import lax
from jax.experimental import pallas as pl
from jax.experimental.pallas import tpu as pltpu

N_DEV = 32


def kernel(x, w_mat):
    m_glob, k_per = x.shape
    k_glob, n = w_mat.shape
    m_per = m_glob // N_DEV

    def body(x_ref, w_ref, out_ref, send_buf, comm_ref, send_sems, recv_sems):
        my = lax.axis_index("i")

        send_buf[...] = x_ref[...].astype(jnp.bfloat16)

        for d in range(1, N_DEV):
            tgt = lax.rem(my + d, N_DEV)
            pltpu.make_async_remote_copy(
                src_ref=send_buf.at[pl.ds(tgt * m_per, m_per), :],
                dst_ref=comm_ref.at[d],
                send_sem=send_sems.at[d],
                recv_sem=recv_sems.at[d],
                device_id=(tgt,),
                device_id_type=pl.DeviceIdType.MESH,
            ).start()

        acc = jnp.dot(
            send_buf[pl.ds(my * m_per, m_per), :],
            w_ref[pl.ds(my * k_per, k_per), :].astype(jnp.bfloat16),
            preferred_element_type=jnp.float32,
        )

        for d in range(1, N_DEV):
            pltpu.make_async_remote_copy(
                src_ref=send_buf.at[pl.ds(d * m_per, m_per), :],
                dst_ref=comm_ref.at[d],
                send_sem=send_sems.at[d],
                recv_sem=recv_sems.at[d],
                device_id=(my,),
                device_id_type=pl.DeviceIdType.MESH,
            ).wait_recv()
            src = lax.rem(my - d + N_DEV, N_DEV)
            acc = acc + jnp.dot(
                comm_ref[d],
                w_ref[pl.ds(src * k_per, k_per), :].astype(jnp.bfloat16),
                preferred_element_type=jnp.float32,
            )

        out_ref[...] = acc * (1.0 / (1.0 + jnp.exp(-acc)))

        for d in range(1, N_DEV):
            tgt = lax.rem(my + d, N_DEV)
            pltpu.make_async_remote_copy(
                src_ref=send_buf.at[pl.ds(tgt * m_per, m_per), :],
                dst_ref=comm_ref.at[d],
                send_sem=send_sems.at[d],
                recv_sem=recv_sems.at[d],
                device_id=(tgt,),
                device_id_type=pl.DeviceIdType.MESH,
            ).wait_send()

    return pl.pallas_call(
        body,
        out_shape=jax.ShapeDtypeStruct((m_per, n), jnp.float32),
        in_specs=[
            pl.BlockSpec(memory_space=pltpu.VMEM),
            pl.BlockSpec(memory_space=pltpu.VMEM),
        ],
        out_specs=pl.BlockSpec(memory_space=pltpu.VMEM),
        scratch_shapes=[
            pltpu.VMEM((m_glob, k_per), jnp.bfloat16),
            pltpu.VMEM((N_DEV, m_per, k_per), jnp.bfloat16),
            pltpu.SemaphoreType.DMA((N_DEV,)),
            pltpu.SemaphoreType.DMA((N_DEV,)),
        ],
        compiler_params=pltpu.CompilerParams(
            vmem_limit_bytes=100 * 1024 * 1024,
        ),
    )(x, w_mat)


# device time: 39277 ns/iter; 1.0839x vs baseline; 1.0839x over previous
import jax
import jax.numpy as jnp
from jax import lax
from jax.experimental import pallas as pl
from jax.experimental.pallas import tpu as pltpu

N_DEV = 32
N_WCHUNKS = 8


def kernel(x, w_mat):
    m_glob, k_per = x.shape
    k_glob, n = w_mat.shape
    m_per = m_glob // N_DEV
    kc = k_glob // N_WCHUNKS

    def body(x_ref, w_hbm, out_ref, send_buf, comm_ref, xrow, wbuf,
             send_sems, recv_sems, local_sems, w_sems):
        my = lax.axis_index("i")

        def w_copy(g, slot):
            return pltpu.make_async_copy(
                w_hbm.at[pl.ds(g * kc, kc), :],
                wbuf.at[slot],
                w_sems.at[slot],
            )

        w_copy(0, 0).start()

        send_buf[...] = x_ref[...].astype(jnp.bfloat16)

        for d in range(1, N_DEV):
            tgt = lax.rem(my + d, N_DEV)
            pltpu.make_async_remote_copy(
                src_ref=send_buf.at[pl.ds(tgt * m_per, m_per), :],
                dst_ref=comm_ref.at[d],
                send_sem=send_sems.at[d],
                recv_sem=recv_sems.at[d],
                device_id=(tgt,),
                device_id_type=pl.DeviceIdType.MESH,
            ).start()
        w_copy(1, 1).start()

        diag = pltpu.make_async_copy(
            send_buf.at[pl.ds(my * m_per, m_per), :],
            xrow.at[:, pl.ds(my * k_per, k_per)],
            local_sems.at[0],
        )
        diag.start()

        for d in range(1, N_DEV):
            pltpu.make_async_remote_copy(
                src_ref=send_buf.at[pl.ds(d * m_per, m_per), :],
                dst_ref=comm_ref.at[d],
                send_sem=send_sems.at[d],
                recv_sem=recv_sems.at[d],
                device_id=(my,),
                device_id_type=pl.DeviceIdType.MESH,
            ).wait_recv()
            src = lax.rem(my - d + N_DEV, N_DEV)
            pltpu.make_async_copy(
                comm_ref.at[d],
                xrow.at[:, pl.ds(src * k_per, k_per)],
                local_sems.at[d],
            ).start()
        diag.wait()
        for d in range(1, N_DEV):
            pltpu.make_async_copy(
                comm_ref.at[d],
                xrow.at[:, pl.ds(d * k_per, k_per)],
                local_sems.at[d],
            ).wait()

        acc = jnp.zeros((m_per, n), jnp.float32)
        for g in range(N_WCHUNKS):
            slot = g % 2
            w_copy(g, slot).wait()
            if g + 2 < N_WCHUNKS:
                w_copy(g + 2, slot).start()
            acc = acc + jnp.dot(
                xrow[:, g * kc:(g + 1) * kc],
                wbuf[slot].astype(jnp.bfloat16),
                preferred_element_type=jnp.float32,
            )
        out_ref[...] = acc * (1.0 / (1.0 + jnp.exp(-acc)))

        for d in range(1, N_DEV):
            tgt = lax.rem(my + d, N_DEV)
            pltpu.make_async_remote_copy(
                src_ref=send_buf.at[pl.ds(tgt * m_per, m_per), :],
                dst_ref=comm_ref.at[d],
                send_sem=send_sems.at[d],
                recv_sem=recv_sems.at[d],
                device_id=(tgt,),
                device_id_type=pl.DeviceIdType.MESH,
            ).wait_send()

    return pl.pallas_call(
        body,
        out_shape=jax.ShapeDtypeStruct((m_per, n), jnp.float32),
        in_specs=[
            pl.BlockSpec(memory_space=pltpu.VMEM),
            pl.BlockSpec(memory_space=pl.ANY),
        ],
        out_specs=pl.BlockSpec(memory_space=pltpu.VMEM),
        scratch_shapes=[
            pltpu.VMEM((m_glob, k_per), jnp.bfloat16),
            pltpu.VMEM((N_DEV, m_per, k_per), jnp.bfloat16),
            pltpu.VMEM((m_per, k_glob), jnp.bfloat16),
            pltpu.VMEM((2, kc, n), jnp.float32),
            pltpu.SemaphoreType.DMA((N_DEV,)),
            pltpu.SemaphoreType.DMA((N_DEV,)),
            pltpu.SemaphoreType.DMA((N_DEV,)),
            pltpu.SemaphoreType.DMA((2,)),
        ],
        compiler_params=pltpu.CompilerParams(
            vmem_limit_bytes=100 * 1024 * 1024,
        ),
    )(x, w_mat)


# device time: 28259 ns/iter; 1.5065x vs baseline; 1.3899x over previous
import jax
import jax.numpy as jnp
from jax import lax
from jax.experimental import pallas as pl
from jax.experimental.pallas import tpu as pltpu

N_DEV = 32
N_GROUPS = 8
TILES_PER_GROUP = N_DEV // N_GROUPS


def kernel(x, w_mat):
    m_glob, k_per = x.shape
    k_glob, n = w_mat.shape
    m_per = m_glob // N_DEV

    def body(x_ref, w_hbm, out_ref, send_buf, xrow, wbuf,
             send_sems, recv_sems, diag_sem, w_sems):
        my = lax.axis_index("i")

        barrier_sem = pltpu.get_barrier_semaphore()
        for d in range(1, N_DEV):
            pl.semaphore_signal(
                barrier_sem, inc=1,
                device_id=lax.rem(my + d, N_DEV),
                device_id_type=pl.DeviceIdType.LOGICAL,
            )

        def w_copy(c):
            return pltpu.make_async_copy(
                w_hbm.at[pl.ds(lax.rem(c + my, N_DEV) * k_per, k_per), :],
                wbuf.at[pl.ds(c * k_per, k_per), :],
                w_sems.at[c],
            )

        group_order = list(range(N_GROUPS))
        w_order = [
            c
            for g in group_order
            for c in range(N_DEV - (g + 1) * TILES_PER_GROUP,
                           N_DEV - g * TILES_PER_GROUP)
        ]
        for c in w_order:
            w_copy(c).start()

        send_buf[...] = x_ref[...].astype(jnp.bfloat16)

        diag = pltpu.make_async_copy(
            send_buf.at[pl.ds(my * m_per, m_per), :],
            xrow.at[:, pl.ds(0, k_per)],
            diag_sem,
        )
        diag.start()

        pl.semaphore_wait(barrier_sem, N_DEV - 1)

        def rdma(d):
            tgt = lax.rem(my + d, N_DEV)
            return pltpu.make_async_remote_copy(
                src_ref=send_buf.at[pl.ds(tgt * m_per, m_per), :],
                dst_ref=xrow.at[:, pl.ds(k_glob - d * k_per, k_per)],
                send_sem=send_sems.at[d],
                recv_sem=recv_sems.at[d],
                device_id=(tgt,),
                device_id_type=pl.DeviceIdType.MESH,
            )

        for d in range(1, N_DEV):
            rdma(d).start()

        kg = TILES_PER_GROUP * k_per
        acc = jnp.zeros((m_per, n), jnp.float32)
        for g in group_order:
            for d in range(g * TILES_PER_GROUP + 1,
                           (g + 1) * TILES_PER_GROUP + 1):
                if d < N_DEV:
                    rdma(d).wait_recv()
                else:
                    diag.wait()
            lo = k_glob - (g + 1) * kg
            for c in range(N_DEV - (g + 1) * TILES_PER_GROUP,
                           N_DEV - g * TILES_PER_GROUP):
                w_copy(c).wait()
            acc = acc + jnp.dot(
                xrow[:, lo:lo + kg],
                wbuf[lo:lo + kg, :].astype(jnp.bfloat16),
                preferred_element_type=jnp.float32,
            )
        out_ref[...] = acc * (1.0 / (1.0 + jnp.exp(-acc)))

        for d in range(1, N_DEV):
            rdma(d).wait_send()

    return pl.pallas_call(
        body,
        out_shape=jax.ShapeDtypeStruct((m_per, n), jnp.float32),
        in_specs=[
            pl.BlockSpec(memory_space=pltpu.VMEM),
            pl.BlockSpec(memory_space=pl.ANY),
        ],
        out_specs=pl.BlockSpec(memory_space=pltpu.VMEM),
        scratch_shapes=[
            pltpu.VMEM((m_glob, k_per), jnp.bfloat16),
            pltpu.VMEM((m_per, k_glob), jnp.bfloat16),
            pltpu.VMEM((k_glob, n), jnp.float32),
            pltpu.SemaphoreType.DMA((N_DEV,)),
            pltpu.SemaphoreType.DMA((N_DEV,)),
            pltpu.SemaphoreType.DMA,
            pltpu.SemaphoreType.DMA((N_DEV,)),
        ],
        compiler_params=pltpu.CompilerParams(
            vmem_limit_bytes=100 * 1024 * 1024,
            collective_id=0,
        ),
    )(x, w_mat)
